# baseline (speedup 1.0000x reference)
import functools

import jax
import jax.numpy as jnp
from jax import lax
from jax.experimental import pallas as pl
from jax.experimental.pallas import tpu as pltpu
from jax.experimental.pallas import tpu_sc as plsc

V = 7
D = 512
B = 16384
NC = 2
NS = 16
NW = NC * NS
B_PER_W = B // NW
CHUNK = 64
N_CHUNKS = B_PER_W // CHUNK
G = 8
SUB = CHUNK // G


def _sc_lookup(idx3d, table):
    mesh = plsc.VectorSubcoreMesh(core_axis_name="c", subcore_axis_name="s")

    @functools.partial(
        pl.kernel,
        mesh=mesh,
        out_type=jax.ShapeDtypeStruct((B, D), jnp.float32),
        scratch_types=[
            pltpu.VMEM((N_CHUNKS * G, SUB), jnp.int32),
            pltpu.VMEM((CHUNK, D), jnp.float32),
            pltpu.VMEM((CHUNK, D), jnp.float32),
            pltpu.SemaphoreType.DMA,
            pltpu.SemaphoreType.DMA,
            pltpu.SemaphoreType.DMA,
            pltpu.SemaphoreType.DMA,
        ],
    )
    def k(idx_hbm, tab_hbm, out_hbm, idx_v, buf0, buf1, g0, g1, s0, s1):
        wid = lax.axis_index("s") * NC + lax.axis_index("c")
        pltpu.sync_copy(idx_hbm.at[wid], idx_v)
        bufs = (buf0, buf1)
        gsem = (g0, g1)
        ssem = (s0, s1)

        def fire(c):
            p = c & 1
            return [
                pltpu.async_copy(
                    tab_hbm.at[idx_v.at[c * G + s]],
                    bufs[p].at[pl.ds(s * SUB, SUB)],
                    gsem[p])
                for s in range(G)
            ]

        gh = [None] * N_CHUNKS
        sh = [None] * N_CHUNKS
        gh[0] = fire(0)
        for c in range(N_CHUNKS):
            p = c & 1
            for h in gh[c]:
                h.wait()
            if c + 1 < N_CHUNKS:
                if c >= 1:
                    sh[c - 1].wait()
                gh[c + 1] = fire(c + 1)
            sh[c] = pltpu.async_copy(
                bufs[p],
                out_hbm.at[pl.ds((wid * N_CHUNKS + c) * CHUNK, CHUNK)],
                ssem[p])
        sh[N_CHUNKS - 2].wait()
        sh[N_CHUNKS - 1].wait()

    return k(idx3d, table)


def kernel(emotion_index, table):
    idx3d = emotion_index.astype(jnp.int32).reshape(NW, N_CHUNKS * G, SUB)
    return _sc_lookup(idx3d, table)


# speedup vs baseline: 2.1481x; 2.1481x over previous
import functools

import jax
import jax.numpy as jnp
from jax import lax
from jax.experimental import pallas as pl
from jax.experimental.pallas import tpu as pltpu
from jax.experimental.pallas import tpu_sc as plsc

V = 7
D = 512
B = 16384
NC = 2
NS = 16
NW = NC * NS
B_PER_W = B // NW
CHUNK = 64
N_CHUNKS = B_PER_W // CHUNK
COLB = D // 16


def _sc_lookup(idx2d, table_flat):
    mesh = plsc.VectorSubcoreMesh(core_axis_name="c", subcore_axis_name="s")

    @functools.partial(
        pl.kernel,
        mesh=mesh,
        out_type=jax.ShapeDtypeStruct((B * D,), jnp.float32),
        scratch_types=[
            pltpu.VMEM((B_PER_W,), jnp.int32),
            pltpu.VMEM((V * D,), jnp.float32),
            pltpu.VMEM((CHUNK * D,), jnp.float32),
            pltpu.VMEM((CHUNK * D,), jnp.float32),
            pltpu.SMEM((B_PER_W,), jnp.int32),
            pltpu.SemaphoreType.DMA,
            pltpu.SemaphoreType.DMA,
        ],
    )
    def k(idx_hbm, tab_hbm, out_hbm, idx_v, tab_v, buf0, buf1, base_s, s0, s1):
        wid = lax.axis_index("s") * NC + lax.axis_index("c")
        pltpu.sync_copy(tab_hbm, tab_v)
        pltpu.sync_copy(idx_hbm.at[wid], idx_v)

        for g in range(B_PER_W // 16):
            iv = idx_v[pl.ds(g * 16, 16)] * D
            for l in range(16):
                base_s[g * 16 + l] = iv[l]

        bufs = (buf0, buf1)
        ssem = (s0, s1)
        sh = [None, None]
        for c in range(N_CHUNKS):
            p = c & 1
            buf = bufs[p]
            if sh[p] is not None:
                sh[p].wait()

            @plsc.parallel_loop(0, CHUNK)
            def row_body(l, buf=buf, c=c):
                base = base_s[c * CHUNK + l]
                for j in range(COLB):
                    buf[pl.ds(l * D + j * 16, 16)] = tab_v[pl.ds(base + j * 16, 16)]

            sh[p] = pltpu.async_copy(
                buf,
                out_hbm.at[pl.ds((wid * B_PER_W + c * CHUNK) * D, CHUNK * D)],
                ssem[p])
        sh[0].wait()
        sh[1].wait()

    return k(idx2d, table_flat)


def kernel(emotion_index, table):
    idx2d = emotion_index.astype(jnp.int32).reshape(NW, B_PER_W)
    out = _sc_lookup(idx2d, table.reshape(V * D))
    return out.reshape(B, D)


# speedup vs baseline: 2.1581x; 1.0047x over previous
import functools

import jax
import jax.numpy as jnp
from jax import lax
from jax.experimental import pallas as pl
from jax.experimental.pallas import tpu as pltpu
from jax.experimental.pallas import tpu_sc as plsc

V = 7
D = 512
B = 16384
NC = 2
NS = 16
NW = NC * NS
B_PER_W = B // NW


def _sc_lookup(idx2d, table_flat):
    mesh = plsc.VectorSubcoreMesh(core_axis_name="c", subcore_axis_name="s")

    @functools.partial(
        pl.kernel,
        mesh=mesh,
        out_type=jax.ShapeDtypeStruct((B * D,), jnp.float32),
        scratch_types=[
            pltpu.VMEM((B_PER_W,), jnp.int32),
            pltpu.VMEM((V * D,), jnp.float32),
            pltpu.SemaphoreType.DMA,
        ],
    )
    def k(idx_hbm, tab_hbm, out_hbm, idx_v, tab_v, sem):
        wid = lax.axis_index("s") * NC + lax.axis_index("c")
        pltpu.sync_copy(tab_hbm, tab_v)
        pltpu.sync_copy(idx_hbm.at[wid], idx_v)
        out_base = wid * (B_PER_W * D)
        handles = []
        for g in range(B_PER_W // 16):
            iv = idx_v[pl.ds(g * 16, 16)] * D
            for l in range(16):
                r = g * 16 + l
                handles.append(pltpu.async_copy(
                    tab_v.at[pl.ds(pl.multiple_of(iv[l], D), D)],
                    out_hbm.at[pl.ds(out_base + r * D, D)],
                    sem))
        for h in handles:
            h.wait()

    return k(idx2d, table_flat)


def kernel(emotion_index, table):
    idx2d = emotion_index.astype(jnp.int32).reshape(NW, B_PER_W)
    out = _sc_lookup(idx2d, table.reshape(V * D))
    return out.reshape(B, D)
